# baseline (speedup 1.0000x reference)
import functools

import jax
import jax.numpy as jnp
from jax import lax
from jax.experimental import pallas as pl
from jax.experimental.pallas import tpu as pltpu
from jax.experimental.pallas import tpu_sc as plsc

B = 16384
DIM = 150
E_ROWS = 1000000
R_ROWS = 1000
NC = 2
NS = 16
NW = NC * NS
RPT = B // NW
CHUNK = 64
NG = RPT // CHUNK
KG = 11
WPR = KG * 16
NL = CHUNK * KG
NSTREAM = (NL + 127) // 128
NFULL = DIM // 16
TAIL_OFF = DIM - 16
E_GMAX = E_ROWS * DIM // 16 - 1
R_GMAX = R_ROWS * DIM // 16 - 1


def _sc_body(gl0, gl1, gl2, gl3, gl4, gl5,
             of0, of1, of2, of3, of4, of5,
             ent_g, rel_g, out_hbm,
             l0, l1, l2, l3, l4, l5,
             v0, v1, v2, v3, v4, v5,
             b0, b1, b2, b3, b4, b5,
             outv, sem):
    wid = lax.axis_index("s") * NC + lax.axis_index("c")
    base = wid * RPT

    gls = (gl0, gl1, gl2, gl3, gl4, gl5)
    ofs = (of0, of1, of2, of3, of4, of5)
    lsts = (l0, l1, l2, l3, l4, l5)
    offv = (v0, v1, v2, v3, v4, v5)
    bufs = (b0, b1, b2, b3, b4, b5)
    tabs = (ent_g, rel_g, ent_g, ent_g, rel_g, ent_g)

    for t in range(6):
        pltpu.sync_copy(ofs[t].at[pl.ds(base, RPT)], offv[t].at[pl.ds(0, RPT)])

    iota16 = lax.broadcasted_iota(jnp.int32, (16,), 0)
    tail_mask = iota16 >= (16 - (DIM - NFULL * 16))

    def compute(o, lv):
        def row(i, lv):
            j = o + i

            def row_slices(t):
                off = offv[t][pl.ds(j, 16)][0]
                bw = i * WPR + off
                w0 = bw + iota16
                rv = w0 >> 4
                cv = w0 & 15
                vals = [plsc.load_gather(bufs[t], [rv + c, cv])
                        for c in range(NFULL)]
                wt = bw + TAIL_OFF + iota16
                vals.append(plsc.load_gather(bufs[t], [wt >> 4, wt & 15]))
                return vals

            ph, pr, pt, nh, nr, nt = (row_slices(t) for t in range(6))
            acc = jnp.zeros((16,), jnp.float32)
            for c in range(NFULL):
                xp = ph[c] + pr[c] - pt[c]
                xn = nh[c] + nr[c] - nt[c]
                acc = acc + (jnp.abs(xp) - jnp.abs(xn))
            xp = ph[NFULL] + pr[NFULL] - pt[NFULL]
            xn = nh[NFULL] + nr[NFULL] - nt[NFULL]
            acc = acc + jnp.where(tail_mask, jnp.abs(xp) - jnp.abs(xn), 0.0)
            for sh in (8, 4, 2, 1):
                acc = acc + jnp.take(acc, (iota16 + sh) & 15)
            hinge = jnp.maximum(acc + 1.0, 0.0)
            return lv + jnp.where(iota16 == 0, hinge, 0.0)

        return lax.fori_loop(0, CHUNK, row, lv)

    loss_vec = jnp.zeros((16,), jnp.float32)
    for g in range(NG):
        o = g * CHUNK
        for t in range(6):
            pltpu.sync_copy(gls[t].at[pl.ds((base + o) * KG, NL)], lsts[t])
        descs = []
        for t in range(6):
            for s in range(NSTREAM):
                so = s * 128
                sl = min(128, NL - so)
                descs.append(pltpu.async_copy(
                    tabs[t].at[lsts[t].at[pl.ds(so, sl)]],
                    bufs[t].at[pl.ds(so, sl)], sem))
        for d in descs:
            d.wait()
        loss_vec = compute(o, loss_vec)

    outv[...] = loss_vec
    pltpu.sync_copy(outv, out_hbm.at[wid])


@functools.partial(
    pl.kernel,
    out_type=jax.ShapeDtypeStruct((NW, 16), jnp.float32),
    mesh=plsc.VectorSubcoreMesh(core_axis_name="c", subcore_axis_name="s",
                                num_cores=NC, num_subcores=NS),
    compiler_params=pltpu.CompilerParams(needs_layout_passes=False,
                                         use_tc_tiling_on_sc=False),
    scratch_types=(
        [pltpu.VMEM((NL,), jnp.int32)] * 6
        + [pltpu.VMEM((RPT + 16,), jnp.int32)] * 6
        + [pltpu.VMEM((NL, 16), jnp.float32)] * 6
        + [pltpu.VMEM((16,), jnp.float32),
           pltpu.SemaphoreType.DMA]),
)
def _sc_partials(*args):
    _sc_body(*args)


def _finish_body(p_ref, o_ref):
    o_ref[...] = jnp.sum(p_ref[...]).reshape(1, 1)


def kernel(pos_h, pos_r, pos_t, neg_h, neg_r, neg_t, ent_emb, rel_emb):
    idxs = [a.astype(jnp.int32) for a in
            (pos_h, pos_r, pos_t, neg_h, neg_r, neg_t)]
    gmaxs = (E_GMAX, R_GMAX, E_GMAX, E_GMAX, R_GMAX, E_GMAX)
    ks = jnp.arange(KG, dtype=jnp.int32)
    gls, ofs = [], []
    for idx, gmax in zip(idxs, gmaxs):
        w = idx * DIM
        gls.append(jnp.minimum((w >> 4)[:, None] + ks, gmax).reshape(-1))
        ofs.append(w & 15)
    ent_g = ent_emb.reshape(-1, 16)
    rel_g = rel_emb.reshape(-1, 16)
    partials = _sc_partials(*gls, *ofs, ent_g, rel_g)
    loss = pl.pallas_call(
        _finish_body,
        out_shape=jax.ShapeDtypeStruct((1, 1), jnp.float32),
    )(partials)
    return loss[0, 0]


# speedup vs baseline: 3.3864x; 3.3864x over previous
import functools

import jax
import jax.numpy as jnp
from jax import lax
from jax.experimental import pallas as pl
from jax.experimental.pallas import tpu as pltpu
from jax.experimental.pallas import tpu_sc as plsc

B = 16384
DIM = 150
MAIN = 128
TAILW = DIM - MAIN
NC = 2
NS = 16
NW = NC * NS
RPT = B // NW
CHUNK = 64
NG = RPT // CHUNK
NMAIN = MAIN // 16


def _sc_body(ph, pr, pt, nh, nr, nt, ent, rel, ent_t, rel_t, out_hbm,
             i0, i1, i2, i3, i4, i5,
             t0, t1, t2, t3, t4, t5,
             m0, m1, m2, m3, m4, m5,
             u0, u1, u2, u3, u4, u5,
             outv, sem):
    wid = lax.axis_index("s") * NC + lax.axis_index("c")
    base = wid * RPT

    idx_hbm = (ph, pr, pt, nh, nr, nt)
    ivs = (i0, i1, i2, i3, i4, i5)
    tls = (t0, t1, t2, t3, t4, t5)
    mbufs = (m0, m1, m2, m3, m4, m5)
    tbufs = (u0, u1, u2, u3, u4, u5)
    tabs = (ent, rel, ent, ent, rel, ent)
    ttabs = (ent_t, rel_t, ent_t, ent_t, rel_t, ent_t)

    for t in range(6):
        pltpu.sync_copy(idx_hbm[t].at[pl.ds(base, RPT)],
                        ivs[t].at[pl.ds(0, RPT)])

    iota16 = lax.broadcasted_iota(jnp.int32, (16,), 0)
    tail_mask = iota16 < (TAILW - 16)

    def compute(o, lv):
        def row(i, lv):
            j = o + i

            def row_slices(t):
                m = ivs[t][pl.ds(j, 16)][0]
                cols = (m & 3) * 32 + iota16
                rows = jnp.full((16,), 0, jnp.int32) + i
                vals = [mbufs[t][i, pl.ds(c * 16, 16)] for c in range(NMAIN)]
                vals.append(plsc.load_gather(tbufs[t], [rows, cols]))
                vals.append(plsc.load_gather(tbufs[t], [rows, cols + 16]))
                return vals

            phv, prv, ptv, nhv, nrv, ntv = (row_slices(t) for t in range(6))
            acc = jnp.zeros((16,), jnp.float32)
            for c in range(NMAIN + 1):
                xp = phv[c] + prv[c] - ptv[c]
                xn = nhv[c] + nrv[c] - ntv[c]
                acc = acc + (jnp.abs(xp) - jnp.abs(xn))
            c = NMAIN + 1
            xp = phv[c] + prv[c] - ptv[c]
            xn = nhv[c] + nrv[c] - ntv[c]
            acc = acc + jnp.where(tail_mask, jnp.abs(xp) - jnp.abs(xn), 0.0)
            for sh in (8, 4, 2, 1):
                acc = acc + jnp.take(acc, (iota16 + sh) & 15)
            hinge = jnp.maximum(acc + 1.0, 0.0)
            return lv + jnp.where(iota16 == 0, hinge, 0.0)

        return lax.fori_loop(0, CHUNK, row, lv)

    def chunk_body(g, lv):
        o = g * CHUNK
        for t in range(6):
            for v in range(CHUNK // 16):
                tls[t][pl.ds(v * 16, 16)] = \
                    lax.shift_right_logical(ivs[t][pl.ds(o + v * 16, 16)], 2)
        descs = []
        for t in range(6):
            descs.append(pltpu.async_copy(
                tabs[t].at[:, pl.ds(0, MAIN)].at[ivs[t].at[pl.ds(o, CHUNK)]],
                mbufs[t], sem))
            descs.append(pltpu.async_copy(
                ttabs[t].at[tls[t]], tbufs[t], sem))
        for d in descs:
            d.wait()
        return compute(o, lv)

    loss_vec = lax.fori_loop(0, NG, chunk_body,
                             jnp.zeros((16,), jnp.float32))

    outv[...] = loss_vec
    pltpu.sync_copy(outv, out_hbm.at[wid])


@functools.partial(
    pl.kernel,
    out_type=jax.ShapeDtypeStruct((NW, 16), jnp.float32),
    mesh=plsc.VectorSubcoreMesh(core_axis_name="c", subcore_axis_name="s",
                                num_cores=NC, num_subcores=NS),
    compiler_params=pltpu.CompilerParams(needs_layout_passes=False,
                                         use_tc_tiling_on_sc=True),
    scratch_types=(
        [pltpu.VMEM((RPT + 16,), jnp.int32)] * 6
        + [pltpu.VMEM((CHUNK,), jnp.int32)] * 6
        + [pltpu.VMEM((CHUNK, MAIN), jnp.float32)] * 6
        + [pltpu.VMEM((CHUNK, MAIN), jnp.float32)] * 6
        + [pltpu.VMEM((16,), jnp.float32),
           pltpu.SemaphoreType.DMA]),
)
def _sc_partials(*args):
    _sc_body(*args)


def _finish_body(p_ref, o_ref):
    o_ref[...] = jnp.sum(p_ref[...]).reshape(1, 1)


def _pack_tails(emb):
    r = emb.shape[0]
    t = jnp.pad(emb[:, MAIN:], ((0, 0), (0, 32 - TAILW)))
    return t.reshape(r // 4, MAIN)


def kernel(pos_h, pos_r, pos_t, neg_h, neg_r, neg_t, ent_emb, rel_emb):
    idxs = [a.astype(jnp.int32) for a in
            (pos_h, pos_r, pos_t, neg_h, neg_r, neg_t)]
    ent_t = _pack_tails(ent_emb)
    rel_t = _pack_tails(rel_emb)
    partials = _sc_partials(*idxs, ent_emb, rel_emb, ent_t, rel_t)
    loss = pl.pallas_call(
        _finish_body,
        out_shape=jax.ShapeDtypeStruct((1, 1), jnp.float32),
    )(partials)
    return loss[0, 0]


# speedup vs baseline: 3.9974x; 1.1804x over previous
import functools

import jax
import jax.numpy as jnp
from jax import lax
from jax.experimental import pallas as pl
from jax.experimental.pallas import tpu as pltpu
from jax.experimental.pallas import tpu_sc as plsc

B = 16384
DIM = 150
MAIN = 128
TAILW = DIM - MAIN
NC = 2
NS = 16
NW = NC * NS
RPT = B // NW
CHUNK = 64
NG = RPT // CHUNK
NMAIN = MAIN // 16


def _sc_body(ph, pr, pt, nh, nr, nt, ent, rel, ent_t, rel_t, out_hbm,
             i0, i1, i2, i3, i4, i5,
             t0, t1, t2, t3, t4, t5,
             m0, m1, m2, m3, m4, m5,
             u0, u1, u2, u3, u4, u5,
             outv, sem):
    wid = lax.axis_index("s") * NC + lax.axis_index("c")
    base = wid * RPT

    idx_hbm = (ph, pr, pt, nh, nr, nt)
    ivs = (i0, i1, i2, i3, i4, i5)
    tls = (t0, t1, t2, t3, t4, t5)
    mbufs = (m0, m1, m2, m3, m4, m5)
    tbufs = (u0, u1, u2, u3, u4, u5)
    tabs = (ent, rel, ent, ent, rel, ent)
    ttabs = (ent_t, rel_t, ent_t, ent_t, rel_t, ent_t)

    for t in range(6):
        pltpu.sync_copy(idx_hbm[t].at[pl.ds(base, RPT)],
                        ivs[t].at[pl.ds(0, RPT)])

    iota16 = lax.broadcasted_iota(jnp.int32, (16,), 0)
    tail_mask = iota16 < (TAILW - 16)

    def compute(o, lv):
        def row(i, lv):
            j = o + i

            def row_slices(t):
                m = ivs[t][pl.ds(j, 16)][0]
                cols = (m & 3) * TAILW + iota16
                rows = jnp.full((16,), 0, jnp.int32) + i
                vals = [mbufs[t][i, pl.ds(c * 16, 16)] for c in range(NMAIN)]
                vals.append(plsc.load_gather(tbufs[t], [rows, cols]))
                vals.append(plsc.load_gather(tbufs[t], [rows, cols + 16]))
                return vals

            phv, prv, ptv, nhv, nrv, ntv = (row_slices(t) for t in range(6))
            acc = jnp.zeros((16,), jnp.float32)
            for c in range(NMAIN + 1):
                xp = phv[c] + prv[c] - ptv[c]
                xn = nhv[c] + nrv[c] - ntv[c]
                acc = acc + (jnp.abs(xp) - jnp.abs(xn))
            c = NMAIN + 1
            xp = phv[c] + prv[c] - ptv[c]
            xn = nhv[c] + nrv[c] - ntv[c]
            acc = acc + jnp.where(tail_mask, jnp.abs(xp) - jnp.abs(xn), 0.0)
            for sh in (8, 4, 2, 1):
                acc = acc + jnp.take(acc, (iota16 + sh) & 15)
            hinge = jnp.maximum(acc + 1.0, 0.0)
            return lv + jnp.where(iota16 == 0, hinge, 0.0)

        return lax.fori_loop(0, CHUNK, row, lv)

    def chunk_body(g, lv):
        o = g * CHUNK
        for t in range(6):
            for v in range(CHUNK // 16):
                tls[t][pl.ds(v * 16, 16)] = \
                    lax.shift_right_logical(ivs[t][pl.ds(o + v * 16, 16)], 2)
        descs = []
        for t in range(6):
            descs.append(pltpu.async_copy(
                tabs[t].at[:, pl.ds(0, MAIN)].at[ivs[t].at[pl.ds(o, CHUNK)]],
                mbufs[t], sem))
            descs.append(pltpu.async_copy(
                ttabs[t].at[tls[t]], tbufs[t], sem))
        for d in descs:
            d.wait()
        return compute(o, lv)

    loss_vec = lax.fori_loop(0, NG, chunk_body,
                             jnp.zeros((16,), jnp.float32))

    outv[...] = loss_vec
    pltpu.sync_copy(outv, out_hbm.at[wid])


@functools.partial(
    pl.kernel,
    out_type=jax.ShapeDtypeStruct((NW, 16), jnp.float32),
    mesh=plsc.VectorSubcoreMesh(core_axis_name="c", subcore_axis_name="s",
                                num_cores=NC, num_subcores=NS),
    compiler_params=pltpu.CompilerParams(needs_layout_passes=False,
                                         use_tc_tiling_on_sc=True),
    scratch_types=(
        [pltpu.VMEM((RPT + 16,), jnp.int32)] * 6
        + [pltpu.VMEM((CHUNK,), jnp.int32)] * 6
        + [pltpu.VMEM((CHUNK, MAIN), jnp.float32)] * 6
        + [pltpu.VMEM((CHUNK, MAIN), jnp.float32)] * 6
        + [pltpu.VMEM((16,), jnp.float32),
           pltpu.SemaphoreType.DMA]),
)
def _sc_partials(*args):
    _sc_body(*args)


def _finish_body(p_ref, o_ref):
    o_ref[...] = jnp.sum(p_ref[...]).reshape(1, 1)


def _pack_tails(emb):
    r = emb.shape[0]
    t = emb[:, MAIN:].reshape(r // 4, 4 * TAILW)
    return jnp.pad(t, ((0, 0), (0, MAIN - 4 * TAILW)))


def kernel(pos_h, pos_r, pos_t, neg_h, neg_r, neg_t, ent_emb, rel_emb):
    idxs = [a.astype(jnp.int32) for a in
            (pos_h, pos_r, pos_t, neg_h, neg_r, neg_t)]
    ent_t = _pack_tails(ent_emb)
    rel_t = _pack_tails(rel_emb)
    partials = _sc_partials(*idxs, ent_emb, rel_emb, ent_t, rel_t)
    loss = pl.pallas_call(
        _finish_body,
        out_shape=jax.ShapeDtypeStruct((1, 1), jnp.float32),
    )(partials)
    return loss[0, 0]
